# baseline (speedup 1.0000x reference)
import functools

import jax
import jax.numpy as jnp
from jax import lax
from jax.experimental import pallas as pl
from jax.experimental.pallas import tpu as pltpu
from jax.experimental.pallas import tpu_sc as plsc

N = 2048
D = 1024
E = 8
K = 2
RD = 256
H = 512
TB = 256
NB = 24
CAP = NB * TB
NW = 32
TPW = N // NW


def _lanes_cumsum8(x):
    for k in (1, 2, 4):
        shifted = jnp.concatenate(
            [jnp.zeros(x.shape[:-1] + (k,), x.dtype), x[..., :-k]], axis=-1)
        x = x + shifted
    return x


def _gelu_exact(x):
    return 0.5 * x * (1.0 + lax.erf(x * 0.7071067811865476))


def _sigmoid(x):
    return 1.0 / (1.0 + jnp.exp(-x))


def _softplus(x):
    return jnp.where(x > 20.0, x, jnp.log(1.0 + jnp.exp(jnp.minimum(x, 20.0))))


def _router_body(h_ref, x_ref, a_ref, b_ref, rb_ref, g_ref, bb_ref,
                 w2_ref, b2_ref, oh1_ref, oh2_ref, rw0_ref, rw1_ref):
    f32 = jnp.float32
    z = (jnp.dot(h_ref[...], a_ref[...], preferred_element_type=f32)
         + jnp.dot(x_ref[...], b_ref[...], preferred_element_type=f32)
         + rb_ref[...])
    m = jnp.mean(z, axis=-1, keepdims=True)
    v = jnp.mean((z - m) * (z - m), axis=-1, keepdims=True)
    z = (z - m) * lax.rsqrt(v + 1e-5) * g_ref[...] + bb_ref[...]
    z = _gelu_exact(z)
    logits = jnp.dot(z, w2_ref[...], preferred_element_type=f32) + b2_ref[...]

    m1 = jnp.max(logits, axis=-1, keepdims=True)
    is1 = (logits == m1).astype(f32)
    first1 = jnp.where(_lanes_cumsum8(is1) - is1 < 0.5, is1, 0.0)
    masked = jnp.where(first1 > 0.5, -jnp.inf, logits)
    m2 = jnp.max(masked, axis=-1, keepdims=True)
    is2 = (masked == m2).astype(f32)
    first2 = jnp.where(_lanes_cumsum8(is2) - is2 < 0.5, is2, 0.0)

    rw0 = 1.0 / (1.0 + jnp.exp(m2 - m1))
    rw1 = 1.0 - rw0
    oh1_ref[...] = first1
    oh2_ref[...] = first2
    rw0_ref[...] = jnp.broadcast_to(rw0, rw0_ref.shape)
    rw1_ref[...] = jnp.broadcast_to(rw1, rw1_ref.shape)


def _router(h, x, a, b, rbias, g, bb, w2, b2):
    f32 = jnp.float32
    TN = 256
    return pl.pallas_call(
        _router_body,
        grid=(N // TN,),
        in_specs=[
            pl.BlockSpec((TN, D), lambda i: (i, 0)),
            pl.BlockSpec((TN, D), lambda i: (i, 0)),
            pl.BlockSpec((D, RD), lambda i: (0, 0)),
            pl.BlockSpec((D, RD), lambda i: (0, 0)),
            pl.BlockSpec((1, RD), lambda i: (0, 0)),
            pl.BlockSpec((1, RD), lambda i: (0, 0)),
            pl.BlockSpec((1, RD), lambda i: (0, 0)),
            pl.BlockSpec((RD, E), lambda i: (0, 0)),
            pl.BlockSpec((1, E), lambda i: (0, 0)),
        ],
        out_specs=(
            pl.BlockSpec((TN, E), lambda i: (i, 0)),
            pl.BlockSpec((TN, E), lambda i: (i, 0)),
            pl.BlockSpec((TN, 128), lambda i: (i, 0)),
            pl.BlockSpec((TN, 128), lambda i: (i, 0)),
        ),
        out_shape=(
            jax.ShapeDtypeStruct((N, E), f32),
            jax.ShapeDtypeStruct((N, E), f32),
            jax.ShapeDtypeStruct((N, 128), f32),
            jax.ShapeDtypeStruct((N, 128), f32),
        ),
    )(h, x, a, b, rbias, g, bb, w2, b2)


def _meta_body(oh1_ref, oh2_ref, slot0_ref, slot1_ref, bexp_ref, bval_ref):
    f32 = jnp.float32
    oh1 = oh1_ref[...]
    oh2 = oh2_ref[...]
    both = oh1 + oh2
    inc = both
    k = 1
    while k < N:
        shifted = jnp.concatenate(
            [jnp.zeros((k, E), f32), inc[:-k, :]], axis=0)
        inc = inc + shifted
        k *= 2
    excl = inc - both
    counts = inc[N - 1:N, :]
    padded = jnp.ceil(counts * (1.0 / TB)) * TB
    offs = _lanes_cumsum8(padded) - padded
    rank0 = jnp.sum(excl * oh1, axis=-1, keepdims=True)
    rank1 = jnp.sum((excl + oh1) * oh2, axis=-1, keepdims=True)
    off0 = jnp.sum(offs * oh1, axis=-1, keepdims=True)
    off1 = jnp.sum(offs * oh2, axis=-1, keepdims=True)
    slot0_ref[...] = (off0 + rank0).astype(jnp.int32)
    slot1_ref[...] = (off1 + rank1).astype(jnp.int32)

    total = jnp.sum(padded, axis=-1, keepdims=True)
    bpos = lax.broadcasted_iota(jnp.int32, (NB, E), 0).astype(f32) * TB
    started = (bpos >= jnp.broadcast_to(offs, (NB, E))).astype(f32)
    bexp = jnp.sum(started, axis=-1, keepdims=True) - 1.0
    bval = bpos[:, 0:1] < jnp.broadcast_to(total, (NB, 1))
    bexp_ref[...] = jnp.clip(bexp, 0.0, E - 1.0).astype(jnp.int32)
    bval_ref[...] = bval.astype(jnp.int32)


def _metadata(oh1, oh2):
    i32 = jnp.int32
    return pl.pallas_call(
        _meta_body,
        out_shape=(
            jax.ShapeDtypeStruct((N, 1), i32),
            jax.ShapeDtypeStruct((N, 1), i32),
            jax.ShapeDtypeStruct((NB, 1), i32),
            jax.ShapeDtypeStruct((NB, 1), i32),
        ),
    )(oh1, oh2)


_CHUNK = 8


def _sc_scatter_body(comb_hbm, slot0_hbm, slot1_hbm, xg_hbm,
                     idx0_v, idx1_v, buf, sem0, sem1):
    info = plsc.get_sparse_core_info()
    nc = info.num_cores
    wid = lax.axis_index("s") * nc + lax.axis_index("c")
    base = wid * TPW
    nch = TPW // _CHUNK
    for r in range(nch):
        pltpu.sync_copy(slot0_hbm.at[pl.ds(base + r * _CHUNK, _CHUNK)],
                        idx0_v.at[r])
        pltpu.sync_copy(slot1_hbm.at[pl.ds(base + r * _CHUNK, _CHUNK)],
                        idx1_v.at[r])
    for c in range(nch):
        t0 = base + c * _CHUNK
        pltpu.sync_copy(comb_hbm.at[pl.ds(t0, _CHUNK)], buf)
        h0 = pltpu.async_copy(buf, xg_hbm.at[idx0_v.at[c]], sem0)
        h1 = pltpu.async_copy(buf, xg_hbm.at[idx1_v.at[c]], sem1)
        h0.wait()
        h1.wait()


def _sc_scatter(comb, slot0, slot1):
    nch = TPW // _CHUNK
    mesh = plsc.VectorSubcoreMesh(core_axis_name="c", subcore_axis_name="s")
    f = functools.partial(
        pl.kernel,
        mesh=mesh,
        out_type=jax.ShapeDtypeStruct((CAP, 2 * D), jnp.float32),
        scratch_types=[
            pltpu.VMEM((nch, _CHUNK), jnp.int32),
            pltpu.VMEM((nch, _CHUNK), jnp.int32),
            pltpu.VMEM((_CHUNK, 2 * D), jnp.float32),
            pltpu.SemaphoreType.DMA,
            pltpu.SemaphoreType.DMA,
        ],
    )(_sc_scatter_body)
    return f(comb, slot0, slot1)


def _sc_gather_body(y_hbm, slot0_hbm, slot1_hbm, z0_hbm, z1_hbm,
                    idx0_v, idx1_v, buf0, buf1, sem0, sem1):
    info = plsc.get_sparse_core_info()
    nc = info.num_cores
    wid = lax.axis_index("s") * nc + lax.axis_index("c")
    base = wid * TPW
    nch = TPW // _CHUNK
    for r in range(nch):
        pltpu.sync_copy(slot0_hbm.at[pl.ds(base + r * _CHUNK, _CHUNK)],
                        idx0_v.at[r])
        pltpu.sync_copy(slot1_hbm.at[pl.ds(base + r * _CHUNK, _CHUNK)],
                        idx1_v.at[r])
    for c in range(nch):
        t0 = base + c * _CHUNK
        g0 = pltpu.async_copy(y_hbm.at[idx0_v.at[c]], buf0, sem0)
        g1 = pltpu.async_copy(y_hbm.at[idx1_v.at[c]], buf1, sem1)
        g0.wait()
        pltpu.sync_copy(buf0, z0_hbm.at[pl.ds(t0, _CHUNK)])
        g1.wait()
        pltpu.sync_copy(buf1, z1_hbm.at[pl.ds(t0, _CHUNK)])


def _sc_gather(y, slot0, slot1):
    nch = TPW // _CHUNK
    mesh = plsc.VectorSubcoreMesh(core_axis_name="c", subcore_axis_name="s")
    f = functools.partial(
        pl.kernel,
        mesh=mesh,
        out_type=(
            jax.ShapeDtypeStruct((N, 4 * D), jnp.float32),
            jax.ShapeDtypeStruct((N, 4 * D), jnp.float32),
        ),
        scratch_types=[
            pltpu.VMEM((nch, _CHUNK), jnp.int32),
            pltpu.VMEM((nch, _CHUNK), jnp.int32),
            pltpu.VMEM((_CHUNK, 4 * D), jnp.float32),
            pltpu.VMEM((_CHUNK, 4 * D), jnp.float32),
            pltpu.SemaphoreType.DMA,
            pltpu.SemaphoreType.DMA,
        ],
    )(_sc_gather_body)
    return f(y, slot0, slot1)


def _mlpq_body(bexp_ref, bval_ref, xg_ref, w1_ref, b1_ref, g_ref, b_ref,
               w2q_ref, b2q_ref, aw_ref, ab_ref, bw_ref, bb_ref,
               gw_ref, gb_ref, vw_ref, vb_ref, y_ref, t_scr):
    i = pl.program_id(0)
    q = pl.program_id(1)

    @pl.when(bval_ref[i] == 1)
    def _():
        f32 = jnp.float32

        @pl.when(q == 0)
        def _():
            t = (jnp.dot(xg_ref[...], w1_ref[0], preferred_element_type=f32)
                 + b1_ref[0])
            m = jnp.mean(t, axis=-1, keepdims=True)
            v = jnp.mean((t - m) * (t - m), axis=-1, keepdims=True)
            t = (t - m) * lax.rsqrt(v + 1e-5) * g_ref[0] + b_ref[0]
            t_scr[...] = _gelu_exact(t)

        t = t_scr[...]
        oq = jnp.dot(t, w2q_ref[0], preferred_element_type=f32) + b2q_ref[0]

        @pl.when(q == 0)
        def _():
            y_ref[...] = _sigmoid(
                jnp.dot(oq, aw_ref[0], preferred_element_type=f32) + ab_ref[0])

        @pl.when(q == 1)
        def _():
            y_ref[...] = _softplus(
                jnp.dot(oq, bw_ref[0], preferred_element_type=f32) + bb_ref[0])

        @pl.when(q == 2)
        def _():
            y_ref[...] = _sigmoid(
                jnp.dot(oq, gw_ref[0], preferred_element_type=f32) + gb_ref[0])

        @pl.when(q == 3)
        def _():
            y_ref[...] = (
                jnp.dot(oq, vw_ref[0], preferred_element_type=f32) + vb_ref[0])


def _grouped_mlp(xg, bexp, bval, e_W1, e_b1, e_ln_g, e_ln_b, e_W2, e_b2,
                 aW, ab, bW, bb, gW, gb, vW, vb):
    f32 = jnp.float32

    def wmap(i, q, bexp, bval):
        return (bexp[i], 0, 0)

    def w2map(i, q, bexp, bval):
        return (bexp[i], 0, q)

    def xmap(i, q, bexp, bval):
        return (i, 0)

    def ymap(i, q, bexp, bval):
        return (i, q)

    return pl.pallas_call(
        _mlpq_body,
        grid_spec=pltpu.PrefetchScalarGridSpec(
            num_scalar_prefetch=2,
            grid=(NB, 4),
            in_specs=[
                pl.BlockSpec((TB, 2 * D), xmap),
                pl.BlockSpec((1, 2 * D, H), wmap),
                pl.BlockSpec((1, 1, H), wmap),
                pl.BlockSpec((1, 1, H), wmap),
                pl.BlockSpec((1, 1, H), wmap),
                pl.BlockSpec((1, H, D), w2map),
                pl.BlockSpec((1, 1, D), w2map),
                pl.BlockSpec((1, D, D), wmap),
                pl.BlockSpec((1, 1, D), wmap),
                pl.BlockSpec((1, D, D), wmap),
                pl.BlockSpec((1, 1, D), wmap),
                pl.BlockSpec((1, D, D), wmap),
                pl.BlockSpec((1, 1, D), wmap),
                pl.BlockSpec((1, D, D), wmap),
                pl.BlockSpec((1, 1, D), wmap),
            ],
            out_specs=pl.BlockSpec((TB, D), ymap),
            scratch_shapes=[pltpu.VMEM((TB, H), f32)],
        ),
        out_shape=jax.ShapeDtypeStruct((CAP, 4 * D), f32),
    )(bexp, bval, xg, e_W1, e_b1, e_ln_g, e_ln_b, e_W2, e_b2,
      aW, ab, bW, bb, gW, gb, vW, vb)


def _combine_body(z0_ref, z1_ref, rw0_ref, rw1_ref, out_ref):
    w0 = rw0_ref[:, 0:1]
    w1 = rw1_ref[:, 0:1]
    out_ref[...] = z0_ref[...] * w0 + z1_ref[...] * w1


def _combine(z0, z1, rw0b, rw1b):
    TN = 256
    return pl.pallas_call(
        _combine_body,
        grid=(N // TN,),
        in_specs=[
            pl.BlockSpec((TN, 4 * D), lambda i: (i, 0)),
            pl.BlockSpec((TN, 4 * D), lambda i: (i, 0)),
            pl.BlockSpec((TN, 128), lambda i: (i, 0)),
            pl.BlockSpec((TN, 128), lambda i: (i, 0)),
        ],
        out_specs=pl.BlockSpec((TN, 4 * D), lambda i: (i, 0)),
        out_shape=jax.ShapeDtypeStruct((N, 4 * D), jnp.float32),
    )(z0, z1, rw0b, rw1b)


def kernel(h, x, layer_idx, layer_emb_table, phase_emb_table, router_W1,
           router_b1, router_ln_g, router_ln_b, router_W2, router_b2,
           e_W1, e_b1, e_ln_g, e_ln_b, e_W2, e_b2,
           aW, ab, bW, bb, gW, gb, vW, vb):
    le = lax.dynamic_slice_in_dim(layer_emb_table, layer_idx, 1, 0)
    pe = phase_emb_table[0:1]
    a = router_W1[:D]
    b = router_W1[D:2 * D]
    rbias = (le @ router_W1[2 * D:2 * D + 32]
             + pe @ router_W1[2 * D + 32:]
             + router_b1.reshape(1, RD))

    oh1, oh2, rw0b, rw1b = _router(
        h, x, a, b, rbias, router_ln_g.reshape(1, RD),
        router_ln_b.reshape(1, RD), router_W2, router_b2.reshape(1, E))

    slot0, slot1, bexp, bval = _metadata(oh1, oh2)
    slot0 = slot0.reshape(N)
    slot1 = slot1.reshape(N)
    bexp = bexp.reshape(NB)
    bval = bval.reshape(NB)

    comb = jnp.concatenate([h, x], axis=-1)
    xg = _sc_scatter(comb, slot0, slot1)

    y = _grouped_mlp(xg, bexp, bval, e_W1, e_b1.reshape(E, 1, H),
                     e_ln_g.reshape(E, 1, H), e_ln_b.reshape(E, 1, H), e_W2,
                     e_b2.reshape(E, 1, 4 * D), aW, ab.reshape(E, 1, D),
                     bW, bb.reshape(E, 1, D), gW, gb.reshape(E, 1, D),
                     vW, vb.reshape(E, 1, D))

    z0, z1 = _sc_gather(y, slot0, slot1)

    out = _combine(z0, z1, rw0b, rw1b)
    alpha = out[:, 0 * D:1 * D]
    beta = out[:, 1 * D:2 * D]
    gate = out[:, 2 * D:3 * D]
    v_cand = out[:, 3 * D:4 * D]
    return (alpha, beta, gate, v_cand)


# speedup vs baseline: 1.1441x; 1.1441x over previous
import functools

import jax
import jax.numpy as jnp
from jax import lax
from jax.experimental import pallas as pl
from jax.experimental.pallas import tpu as pltpu
from jax.experimental.pallas import tpu_sc as plsc

N = 2048
D = 1024
E = 8
K = 2
RD = 256
H = 512
TB = 256
NB = 24
CAP = NB * TB
NW = 32
TPW = N // NW


def _lanes_cumsum8(x):
    for k in (1, 2, 4):
        shifted = jnp.concatenate(
            [jnp.zeros(x.shape[:-1] + (k,), x.dtype), x[..., :-k]], axis=-1)
        x = x + shifted
    return x


def _gelu_exact(x):
    return 0.5 * x * (1.0 + lax.erf(x * 0.7071067811865476))


def _sigmoid(x):
    return 1.0 / (1.0 + jnp.exp(-x))


def _softplus(x):
    return jnp.where(x > 20.0, x, jnp.log(1.0 + jnp.exp(jnp.minimum(x, 20.0))))


def _router_body(h_ref, x_ref, a_ref, b_ref, rb_ref, g_ref, bb_ref,
                 w2_ref, b2_ref, oh1_ref, oh2_ref, w0_ref, w1_ref):
    f32 = jnp.float32
    z = (jnp.dot(h_ref[...], a_ref[...], preferred_element_type=f32)
         + jnp.dot(x_ref[...], b_ref[...], preferred_element_type=f32)
         + rb_ref[...])
    m = jnp.mean(z, axis=-1, keepdims=True)
    v = jnp.mean((z - m) * (z - m), axis=-1, keepdims=True)
    z = (z - m) * lax.rsqrt(v + 1e-5) * g_ref[...] + bb_ref[...]
    z = _gelu_exact(z)
    logits = jnp.dot(z, w2_ref[...], preferred_element_type=f32) + b2_ref[...]

    m1 = jnp.max(logits, axis=-1, keepdims=True)
    is1 = (logits == m1).astype(f32)
    first1 = jnp.where(_lanes_cumsum8(is1) - is1 < 0.5, is1, 0.0)
    masked = jnp.where(first1 > 0.5, -jnp.inf, logits)
    m2 = jnp.max(masked, axis=-1, keepdims=True)
    is2 = (masked == m2).astype(f32)
    first2 = jnp.where(_lanes_cumsum8(is2) - is2 < 0.5, is2, 0.0)

    rw0 = 1.0 / (1.0 + jnp.exp(m2 - m1))
    rw1 = 1.0 - rw0
    oh1_ref[...] = first1
    oh2_ref[...] = first2
    w0_ref[...] = jnp.broadcast_to(rw0, w0_ref.shape)
    w1_ref[...] = jnp.broadcast_to(rw1, w1_ref.shape)


def _router(h, x, a, b, rbias, g, bb, w2, b2):
    f32 = jnp.float32
    TN = 256
    return pl.pallas_call(
        _router_body,
        grid=(N // TN,),
        in_specs=[
            pl.BlockSpec((TN, D), lambda i: (i, 0)),
            pl.BlockSpec((TN, D), lambda i: (i, 0)),
            pl.BlockSpec((D, RD), lambda i: (0, 0)),
            pl.BlockSpec((D, RD), lambda i: (0, 0)),
            pl.BlockSpec((1, RD), lambda i: (0, 0)),
            pl.BlockSpec((1, RD), lambda i: (0, 0)),
            pl.BlockSpec((1, RD), lambda i: (0, 0)),
            pl.BlockSpec((RD, E), lambda i: (0, 0)),
            pl.BlockSpec((1, E), lambda i: (0, 0)),
        ],
        out_specs=(
            pl.BlockSpec((TN, E), lambda i: (i, 0)),
            pl.BlockSpec((TN, E), lambda i: (i, 0)),
            pl.BlockSpec((TN, 16), lambda i: (i, 0)),
            pl.BlockSpec((TN, 16), lambda i: (i, 0)),
        ),
        out_shape=(
            jax.ShapeDtypeStruct((N, E), f32),
            jax.ShapeDtypeStruct((N, E), f32),
            jax.ShapeDtypeStruct((N, 16), f32),
            jax.ShapeDtypeStruct((N, 16), f32),
        ),
    )(h, x, a, b, rbias, g, bb, w2, b2)


def _meta_body(oh1_ref, oh2_ref, slot0_ref, slot1_ref, bexp_ref, bval_ref):
    f32 = jnp.float32
    oh1 = oh1_ref[...]
    oh2 = oh2_ref[...]
    both = oh1 + oh2
    inc = both
    k = 1
    while k < N:
        shifted = jnp.concatenate(
            [jnp.zeros((k, E), f32), inc[:-k, :]], axis=0)
        inc = inc + shifted
        k *= 2
    excl = inc - both
    counts = inc[N - 1:N, :]
    padded = jnp.ceil(counts * (1.0 / TB)) * TB
    offs = _lanes_cumsum8(padded) - padded
    rank0 = jnp.sum(excl * oh1, axis=-1, keepdims=True)
    rank1 = jnp.sum((excl + oh1) * oh2, axis=-1, keepdims=True)
    off0 = jnp.sum(offs * oh1, axis=-1, keepdims=True)
    off1 = jnp.sum(offs * oh2, axis=-1, keepdims=True)
    slot0_ref[...] = (off0 + rank0).astype(jnp.int32)
    slot1_ref[...] = (off1 + rank1).astype(jnp.int32)

    total = jnp.sum(padded, axis=-1, keepdims=True)
    bpos = lax.broadcasted_iota(jnp.int32, (NB, E), 0).astype(f32) * TB
    started = (bpos >= jnp.broadcast_to(offs, (NB, E))).astype(f32)
    bexp = jnp.sum(started, axis=-1, keepdims=True) - 1.0
    bval = bpos[:, 0:1] < jnp.broadcast_to(total, (NB, 1))
    bexp_ref[...] = jnp.clip(bexp, 0.0, E - 1.0).astype(jnp.int32)
    bval_ref[...] = bval.astype(jnp.int32)


def _metadata(oh1, oh2):
    i32 = jnp.int32
    return pl.pallas_call(
        _meta_body,
        out_shape=(
            jax.ShapeDtypeStruct((N, 1), i32),
            jax.ShapeDtypeStruct((N, 1), i32),
            jax.ShapeDtypeStruct((NB, 1), i32),
            jax.ShapeDtypeStruct((NB, 1), i32),
        ),
    )(oh1, oh2)


_SCH = 16
_GCH = 4


def _sc_scatter_body(h_hbm, x_hbm, slot0_hbm, slot1_hbm, xgh_hbm, xgx_hbm,
                     idx0_v, idx1_v, bufh0, bufh1, bufx0, bufx1, sem):
    info = plsc.get_sparse_core_info()
    nc = info.num_cores
    wid = lax.axis_index("s") * nc + lax.axis_index("c")
    base = wid * TPW
    nch = TPW // _SCH
    for r in range(nch):
        pltpu.sync_copy(slot0_hbm.at[pl.ds(base + r * _SCH, _SCH)],
                        idx0_v.at[r])
        pltpu.sync_copy(slot1_hbm.at[pl.ds(base + r * _SCH, _SCH)],
                        idx1_v.at[r])
    bufh = (bufh0, bufh1)
    bufx = (bufx0, bufx1)
    pend = [None, None]
    for c in range(nch):
        t0 = base + c * _SCH
        pb = pend[c % 2]
        if pb is not None:
            for hnd in pb:
                hnd.wait()
        bh = bufh[c % 2]
        bx = bufx[c % 2]
        pltpu.sync_copy(h_hbm.at[pl.ds(t0, _SCH)], bh)
        pltpu.sync_copy(x_hbm.at[pl.ds(t0, _SCH)], bx)
        pend[c % 2] = (
            pltpu.async_copy(bh, xgh_hbm.at[idx0_v.at[c]], sem),
            pltpu.async_copy(bh, xgh_hbm.at[idx1_v.at[c]], sem),
            pltpu.async_copy(bx, xgx_hbm.at[idx0_v.at[c]], sem),
            pltpu.async_copy(bx, xgx_hbm.at[idx1_v.at[c]], sem),
        )
    for pb in pend:
        if pb is not None:
            for hnd in pb:
                hnd.wait()


def _sc_scatter(h, x, slot0, slot1):
    nch = TPW // _SCH
    f32 = jnp.float32
    mesh = plsc.VectorSubcoreMesh(core_axis_name="c", subcore_axis_name="s")
    f = functools.partial(
        pl.kernel,
        mesh=mesh,
        out_type=(
            jax.ShapeDtypeStruct((CAP, D), f32),
            jax.ShapeDtypeStruct((CAP, D), f32),
        ),
        scratch_types=[
            pltpu.VMEM((nch, _SCH), jnp.int32),
            pltpu.VMEM((nch, _SCH), jnp.int32),
            pltpu.VMEM((_SCH, D), f32),
            pltpu.VMEM((_SCH, D), f32),
            pltpu.VMEM((_SCH, D), f32),
            pltpu.VMEM((_SCH, D), f32),
            pltpu.SemaphoreType.DMA,
        ],
    )(_sc_scatter_body)
    return f(h, x, slot0, slot1)


def _mlp1_body(bexp_ref, bval_ref, xgh_ref, xgx_ref, w1_ref, b1_ref, g_ref,
               b_ref, w2_ref, b2_ref, o_ref):
    i = pl.program_id(0)

    @pl.when(bval_ref[i] == 1)
    def _():
        f32 = jnp.float32
        w1 = w1_ref[0]
        t = (jnp.dot(xgh_ref[...], w1[:D], preferred_element_type=f32)
             + jnp.dot(xgx_ref[...], w1[D:], preferred_element_type=f32)
             + b1_ref[0])
        m = jnp.mean(t, axis=-1, keepdims=True)
        v = jnp.mean((t - m) * (t - m), axis=-1, keepdims=True)
        t = (t - m) * lax.rsqrt(v + 1e-5) * g_ref[0] + b_ref[0]
        t = _gelu_exact(t)
        o_ref[...] = (jnp.dot(t, w2_ref[0], preferred_element_type=f32)
                      + b2_ref[0])


def _mlp2_body(bexp_ref, bval_ref, o_ref, aw_ref, ab_ref, bw_ref, bb_ref,
               gw_ref, gb_ref, vw_ref, vb_ref, y_ref):
    i = pl.program_id(0)

    @pl.when(bval_ref[i] == 1)
    def _():
        f32 = jnp.float32
        o = o_ref[...]
        a = (jnp.dot(o[:, 0 * D:1 * D], aw_ref[0], preferred_element_type=f32)
             + ab_ref[0])
        b = (jnp.dot(o[:, 1 * D:2 * D], bw_ref[0], preferred_element_type=f32)
             + bb_ref[0])
        g = (jnp.dot(o[:, 2 * D:3 * D], gw_ref[0], preferred_element_type=f32)
             + gb_ref[0])
        v = (jnp.dot(o[:, 3 * D:4 * D], vw_ref[0], preferred_element_type=f32)
             + vb_ref[0])
        y_ref[:, 0 * D:1 * D] = _sigmoid(a)
        y_ref[:, 1 * D:2 * D] = _softplus(b)
        y_ref[:, 2 * D:3 * D] = _sigmoid(g)
        y_ref[:, 3 * D:4 * D] = v


def _grouped_mlp(xgh, xgx, bexp, bval, e_W1, e_b1, e_ln_g, e_ln_b, e_W2,
                 e_b2, aW, ab, bW, bb, gW, gb, vW, vb):
    f32 = jnp.float32

    def wmap(i, bexp, bval):
        return (bexp[i], 0, 0)

    def xmap(i, bexp, bval):
        return (i, 0)

    o = pl.pallas_call(
        _mlp1_body,
        grid_spec=pltpu.PrefetchScalarGridSpec(
            num_scalar_prefetch=2,
            grid=(NB,),
            in_specs=[
                pl.BlockSpec((TB, D), xmap),
                pl.BlockSpec((TB, D), xmap),
                pl.BlockSpec((1, 2 * D, H), wmap),
                pl.BlockSpec((1, 1, H), wmap),
                pl.BlockSpec((1, 1, H), wmap),
                pl.BlockSpec((1, 1, H), wmap),
                pl.BlockSpec((1, H, 4 * D), wmap),
                pl.BlockSpec((1, 1, 4 * D), wmap),
            ],
            out_specs=pl.BlockSpec((TB, 4 * D), xmap),
        ),
        out_shape=jax.ShapeDtypeStruct((CAP, 4 * D), f32),
    )(bexp, bval, xgh, xgx, e_W1, e_b1, e_ln_g, e_ln_b, e_W2, e_b2)

    y = pl.pallas_call(
        _mlp2_body,
        grid_spec=pltpu.PrefetchScalarGridSpec(
            num_scalar_prefetch=2,
            grid=(NB,),
            in_specs=[
                pl.BlockSpec((TB, 4 * D), xmap),
                pl.BlockSpec((1, D, D), wmap),
                pl.BlockSpec((1, 1, D), wmap),
                pl.BlockSpec((1, D, D), wmap),
                pl.BlockSpec((1, 1, D), wmap),
                pl.BlockSpec((1, D, D), wmap),
                pl.BlockSpec((1, 1, D), wmap),
                pl.BlockSpec((1, D, D), wmap),
                pl.BlockSpec((1, 1, D), wmap),
            ],
            out_specs=pl.BlockSpec((TB, 4 * D), xmap),
        ),
        out_shape=jax.ShapeDtypeStruct((CAP, 4 * D), f32),
    )(bexp, bval, o, aW, ab, bW, bb, gW, gb, vW, vb)
    return y


def _sc_combine_body(y_hbm, slot0_hbm, slot1_hbm, w0_hbm, w1_hbm,
                     oa_hbm, ob_hbm, og_hbm, ov_hbm,
                     idx0_v, idx1_v, w0_v, w1_v,
                     g0a, g0b, g1a, g1b, q0, q1, q2, q3, sem0, sem1):
    info = plsc.get_sparse_core_info()
    nc = info.num_cores
    wid = lax.axis_index("s") * nc + lax.axis_index("c")
    base = wid * TPW
    nch = TPW // _GCH
    for r in range(TPW // 8):
        pltpu.sync_copy(slot0_hbm.at[pl.ds(base + r * 8, 8)],
                        idx0_v.at[r])
        pltpu.sync_copy(slot1_hbm.at[pl.ds(base + r * 8, 8)],
                        idx1_v.at[r])
    pltpu.sync_copy(w0_hbm.at[pl.ds(base, TPW)], w0_v)
    pltpu.sync_copy(w1_hbm.at[pl.ds(base, TPW)], w1_v)

    buf0 = (g0a, g0b)
    buf1 = (g1a, g1b)
    quarters = (q0, q1, q2, q3)
    outs = (oa_hbm, ob_hbm, og_hbm, ov_hbm)

    def issue(c, par):
        i0 = idx0_v.at[c // 2, pl.ds((c % 2) * _GCH, _GCH)]
        i1 = idx1_v.at[c // 2, pl.ds((c % 2) * _GCH, _GCH)]
        return (pltpu.async_copy(y_hbm.at[i0], buf0[par], sem0),
                pltpu.async_copy(y_hbm.at[i1], buf1[par], sem1))

    pend = [None, None]
    pend[0] = issue(0, 0)
    for c in range(nch):
        if c + 1 < nch:
            pend[(c + 1) % 2] = issue(c + 1, (c + 1) % 2)
        for hnd in pend[c % 2]:
            hnd.wait()
        b0 = buf0[c % 2]
        b1 = buf1[c % 2]
        for t in range(_GCH):
            w0s = w0_v[c * _GCH + t]
            w1s = w1_v[c * _GCH + t]
            for q in range(4):
                qref = quarters[q]

                def body(j, carry, b0=b0, b1=b1, qref=qref, t=t, q=q,
                         w0s=w0s, w1s=w1s):
                    off = q * D + j * 16
                    r0 = b0[t, pl.ds(off, 16)]
                    r1 = b1[t, pl.ds(off, 16)]
                    qref[t, pl.ds(j * 16, 16)] = r0 * w0s + r1 * w1s
                    return carry

                lax.fori_loop(0, D // 16, body, 0)
        t0 = base + c * _GCH
        for q in range(4):
            pltpu.sync_copy(quarters[q], outs[q].at[pl.ds(t0, _GCH)])


def _sc_combine(y, slot0, slot1, w0, w1):
    f32 = jnp.float32
    mesh = plsc.VectorSubcoreMesh(core_axis_name="c", subcore_axis_name="s")
    f = functools.partial(
        pl.kernel,
        mesh=mesh,
        out_type=(
            jax.ShapeDtypeStruct((N, D), f32),
            jax.ShapeDtypeStruct((N, D), f32),
            jax.ShapeDtypeStruct((N, D), f32),
            jax.ShapeDtypeStruct((N, D), f32),
        ),
        scratch_types=[
            pltpu.VMEM((TPW // 8, 8), jnp.int32),
            pltpu.VMEM((TPW // 8, 8), jnp.int32),
            pltpu.VMEM((TPW, 16), f32),
            pltpu.VMEM((TPW, 16), f32),
            pltpu.VMEM((_GCH, 4 * D), f32),
            pltpu.VMEM((_GCH, 4 * D), f32),
            pltpu.VMEM((_GCH, 4 * D), f32),
            pltpu.VMEM((_GCH, 4 * D), f32),
            pltpu.VMEM((_GCH, D), f32),
            pltpu.VMEM((_GCH, D), f32),
            pltpu.VMEM((_GCH, D), f32),
            pltpu.VMEM((_GCH, D), f32),
            pltpu.SemaphoreType.DMA,
            pltpu.SemaphoreType.DMA,
        ],
    )(_sc_combine_body)
    return f(y, slot0, slot1, w0, w1)


def kernel(h, x, layer_idx, layer_emb_table, phase_emb_table, router_W1,
           router_b1, router_ln_g, router_ln_b, router_W2, router_b2,
           e_W1, e_b1, e_ln_g, e_ln_b, e_W2, e_b2,
           aW, ab, bW, bb, gW, gb, vW, vb):
    le = lax.dynamic_slice_in_dim(layer_emb_table, layer_idx, 1, 0)
    pe = phase_emb_table[0:1]
    a = router_W1[:D]
    b = router_W1[D:2 * D]
    rbias = (le @ router_W1[2 * D:2 * D + 32]
             + pe @ router_W1[2 * D + 32:]
             + router_b1.reshape(1, RD))

    oh1, oh2, w0, w1 = _router(
        h, x, a, b, rbias, router_ln_g.reshape(1, RD),
        router_ln_b.reshape(1, RD), router_W2, router_b2.reshape(1, E))

    slot0, slot1, bexp, bval = _metadata(oh1, oh2)
    slot0 = slot0.reshape(N)
    slot1 = slot1.reshape(N)
    bexp = bexp.reshape(NB)
    bval = bval.reshape(NB)

    xgh, xgx = _sc_scatter(h, x, slot0, slot1)

    y = _grouped_mlp(xgh, xgx, bexp, bval, e_W1, e_b1.reshape(E, 1, H),
                     e_ln_g.reshape(E, 1, H), e_ln_b.reshape(E, 1, H), e_W2,
                     e_b2.reshape(E, 1, 4 * D), aW, ab.reshape(E, 1, D),
                     bW, bb.reshape(E, 1, D), gW, gb.reshape(E, 1, D),
                     vW, vb.reshape(E, 1, D))

    alpha, beta, gate, v_cand = _sc_combine(y, slot0, slot1, w0, w1)
    return (alpha, beta, gate, v_cand)


# speedup vs baseline: 1.2322x; 1.0769x over previous
import functools

import jax
import jax.numpy as jnp
from jax import lax
from jax.experimental import pallas as pl
from jax.experimental.pallas import tpu as pltpu
from jax.experimental.pallas import tpu_sc as plsc

N = 2048
D = 1024
E = 8
K = 2
RD = 256
H = 512
TB = 256
NB = 24
CAP = NB * TB
NW = 32
TPW = N // NW


def _lanes_cumsum8(x):
    for k in (1, 2, 4):
        shifted = jnp.concatenate(
            [jnp.zeros(x.shape[:-1] + (k,), x.dtype), x[..., :-k]], axis=-1)
        x = x + shifted
    return x


def _gelu_exact(x):
    return 0.5 * x * (1.0 + lax.erf(x * 0.7071067811865476))


def _sigmoid(x):
    return 1.0 / (1.0 + jnp.exp(-x))


def _softplus(x):
    return jnp.where(x > 20.0, x, jnp.log(1.0 + jnp.exp(jnp.minimum(x, 20.0))))


def _router_body(h_ref, x_ref, a_ref, b_ref, rb_ref, g_ref, bb_ref,
                 w2_ref, b2_ref, oh1_ref, oh2_ref, w0_ref, w1_ref):
    f32 = jnp.float32
    z = (jnp.dot(h_ref[...], a_ref[...], preferred_element_type=f32)
         + jnp.dot(x_ref[...], b_ref[...], preferred_element_type=f32)
         + rb_ref[...])
    m = jnp.mean(z, axis=-1, keepdims=True)
    v = jnp.mean((z - m) * (z - m), axis=-1, keepdims=True)
    z = (z - m) * lax.rsqrt(v + 1e-5) * g_ref[...] + bb_ref[...]
    z = _gelu_exact(z)
    logits = jnp.dot(z, w2_ref[...], preferred_element_type=f32) + b2_ref[...]

    m1 = jnp.max(logits, axis=-1, keepdims=True)
    is1 = (logits == m1).astype(f32)
    first1 = jnp.where(_lanes_cumsum8(is1) - is1 < 0.5, is1, 0.0)
    masked = jnp.where(first1 > 0.5, -jnp.inf, logits)
    m2 = jnp.max(masked, axis=-1, keepdims=True)
    is2 = (masked == m2).astype(f32)
    first2 = jnp.where(_lanes_cumsum8(is2) - is2 < 0.5, is2, 0.0)

    rw0 = 1.0 / (1.0 + jnp.exp(m2 - m1))
    rw1 = 1.0 - rw0
    oh1_ref[...] = first1
    oh2_ref[...] = first2
    w0_ref[...] = jnp.broadcast_to(rw0, w0_ref.shape)
    w1_ref[...] = jnp.broadcast_to(rw1, w1_ref.shape)


def _router(h, x, a, b, rbias, g, bb, w2, b2):
    f32 = jnp.float32
    TN = 256
    return pl.pallas_call(
        _router_body,
        grid=(N // TN,),
        in_specs=[
            pl.BlockSpec((TN, D), lambda i: (i, 0)),
            pl.BlockSpec((TN, D), lambda i: (i, 0)),
            pl.BlockSpec((D, RD), lambda i: (0, 0)),
            pl.BlockSpec((D, RD), lambda i: (0, 0)),
            pl.BlockSpec((1, RD), lambda i: (0, 0)),
            pl.BlockSpec((1, RD), lambda i: (0, 0)),
            pl.BlockSpec((1, RD), lambda i: (0, 0)),
            pl.BlockSpec((RD, E), lambda i: (0, 0)),
            pl.BlockSpec((1, E), lambda i: (0, 0)),
        ],
        out_specs=(
            pl.BlockSpec((TN, E), lambda i: (i, 0)),
            pl.BlockSpec((TN, E), lambda i: (i, 0)),
            pl.BlockSpec((TN, 16), lambda i: (i, 0)),
            pl.BlockSpec((TN, 16), lambda i: (i, 0)),
        ),
        out_shape=(
            jax.ShapeDtypeStruct((N, E), f32),
            jax.ShapeDtypeStruct((N, E), f32),
            jax.ShapeDtypeStruct((N, 16), f32),
            jax.ShapeDtypeStruct((N, 16), f32),
        ),
    )(h, x, a, b, rbias, g, bb, w2, b2)


def _meta_body(oh1_ref, oh2_ref, slot0_ref, slot1_ref, bexp_ref, bval_ref):
    f32 = jnp.float32
    oh1 = oh1_ref[...]
    oh2 = oh2_ref[...]
    both = oh1 + oh2
    inc = both
    k = 1
    while k < N:
        shifted = jnp.concatenate(
            [jnp.zeros((k, E), f32), inc[:-k, :]], axis=0)
        inc = inc + shifted
        k *= 2
    excl = inc - both
    counts = inc[N - 1:N, :]
    padded = jnp.ceil(counts * (1.0 / TB)) * TB
    offs = _lanes_cumsum8(padded) - padded
    rank0 = jnp.sum(excl * oh1, axis=-1, keepdims=True)
    rank1 = jnp.sum((excl + oh1) * oh2, axis=-1, keepdims=True)
    off0 = jnp.sum(offs * oh1, axis=-1, keepdims=True)
    off1 = jnp.sum(offs * oh2, axis=-1, keepdims=True)
    slot0_ref[...] = (off0 + rank0).astype(jnp.int32)
    slot1_ref[...] = (off1 + rank1).astype(jnp.int32)

    total = jnp.sum(padded, axis=-1, keepdims=True)
    bpos = lax.broadcasted_iota(jnp.int32, (NB, E), 0).astype(f32) * TB
    started = (bpos >= jnp.broadcast_to(offs, (NB, E))).astype(f32)
    bexp = jnp.sum(started, axis=-1, keepdims=True) - 1.0
    bval = bpos[:, 0:1] < jnp.broadcast_to(total, (NB, 1))
    bexp_ref[...] = jnp.clip(bexp, 0.0, E - 1.0).astype(jnp.int32)
    bval_ref[...] = bval.astype(jnp.int32)


def _metadata(oh1, oh2):
    i32 = jnp.int32
    return pl.pallas_call(
        _meta_body,
        out_shape=(
            jax.ShapeDtypeStruct((N, 1), i32),
            jax.ShapeDtypeStruct((N, 1), i32),
            jax.ShapeDtypeStruct((NB, 1), i32),
            jax.ShapeDtypeStruct((NB, 1), i32),
        ),
    )(oh1, oh2)


_SCH = 16
_GCH = 4


def _sc_scatter_body(h_hbm, x_hbm, slot0_hbm, slot1_hbm, xgh_hbm, xgx_hbm,
                     idx0_v, idx1_v, bufh0, bufh1, bufx0, bufx1, sem):
    info = plsc.get_sparse_core_info()
    nc = info.num_cores
    wid = lax.axis_index("s") * nc + lax.axis_index("c")
    base = wid * TPW
    nch = TPW // _SCH
    pltpu.sync_copy(slot0_hbm.at[wid], idx0_v)
    pltpu.sync_copy(slot1_hbm.at[wid], idx1_v)
    bufh = (bufh0, bufh1)
    bufx = (bufx0, bufx1)
    pend = [None, None]
    for c in range(nch):
        t0 = base + c * _SCH
        pb = pend[c % 2]
        if pb is not None:
            for hnd in pb:
                hnd.wait()
        bh = bufh[c % 2]
        bx = bufx[c % 2]
        pltpu.sync_copy(h_hbm.at[pl.ds(t0, _SCH)], bh)
        pltpu.sync_copy(x_hbm.at[pl.ds(t0, _SCH)], bx)
        pend[c % 2] = (
            pltpu.async_copy(bh, xgh_hbm.at[idx0_v.at[c]], sem),
            pltpu.async_copy(bh, xgh_hbm.at[idx1_v.at[c]], sem),
            pltpu.async_copy(bx, xgx_hbm.at[idx0_v.at[c]], sem),
            pltpu.async_copy(bx, xgx_hbm.at[idx1_v.at[c]], sem),
        )
    for pb in pend:
        if pb is not None:
            for hnd in pb:
                hnd.wait()


def _sc_scatter(h, x, slot0, slot1):
    nch = TPW // _SCH
    f32 = jnp.float32
    mesh = plsc.VectorSubcoreMesh(core_axis_name="c", subcore_axis_name="s")
    f = functools.partial(
        pl.kernel,
        mesh=mesh,
        out_type=(
            jax.ShapeDtypeStruct((CAP, D), f32),
            jax.ShapeDtypeStruct((CAP, D), f32),
        ),
        scratch_types=[
            pltpu.VMEM((nch, _SCH), jnp.int32),
            pltpu.VMEM((nch, _SCH), jnp.int32),
            pltpu.VMEM((_SCH, D), f32),
            pltpu.VMEM((_SCH, D), f32),
            pltpu.VMEM((_SCH, D), f32),
            pltpu.VMEM((_SCH, D), f32),
            pltpu.SemaphoreType.DMA,
        ],
    )(_sc_scatter_body)
    return f(h, x, slot0, slot1)


def _mlp1_body(bexp_ref, bval_ref, xgh_ref, xgx_ref, w1_ref, b1_ref, g_ref,
               b_ref, w2_ref, b2_ref, o_ref):
    i = pl.program_id(0)

    @pl.when(bval_ref[i] == 1)
    def _():
        f32 = jnp.float32
        w1 = w1_ref[0]
        t = (jnp.dot(xgh_ref[...], w1[:D], preferred_element_type=f32)
             + jnp.dot(xgx_ref[...], w1[D:], preferred_element_type=f32)
             + b1_ref[0])
        m = jnp.mean(t, axis=-1, keepdims=True)
        v = jnp.mean((t - m) * (t - m), axis=-1, keepdims=True)
        t = (t - m) * lax.rsqrt(v + 1e-5) * g_ref[0] + b_ref[0]
        t = _gelu_exact(t)
        o_ref[...] = (jnp.dot(t, w2_ref[0], preferred_element_type=f32)
                      + b2_ref[0]).astype(jnp.bfloat16)


def _mlp2_body(bexp_ref, bval_ref, o_ref, aw_ref, ab_ref, bw_ref, bb_ref,
               gw_ref, gb_ref, vw_ref, vb_ref, y_ref):
    i = pl.program_id(0)

    @pl.when(bval_ref[i] == 1)
    def _():
        f32 = jnp.float32
        o = o_ref[...].astype(f32)
        a = (jnp.dot(o[:, 0 * D:1 * D], aw_ref[0], preferred_element_type=f32)
             + ab_ref[0])
        b = (jnp.dot(o[:, 1 * D:2 * D], bw_ref[0], preferred_element_type=f32)
             + bb_ref[0])
        g = (jnp.dot(o[:, 2 * D:3 * D], gw_ref[0], preferred_element_type=f32)
             + gb_ref[0])
        v = (jnp.dot(o[:, 3 * D:4 * D], vw_ref[0], preferred_element_type=f32)
             + vb_ref[0])
        y_ref[:, 0 * D:1 * D] = _sigmoid(a)
        y_ref[:, 1 * D:2 * D] = _softplus(b)
        y_ref[:, 2 * D:3 * D] = _sigmoid(g)
        y_ref[:, 3 * D:4 * D] = v


def _grouped_mlp(xgh, xgx, bexp, bval, e_W1, e_b1, e_ln_g, e_ln_b, e_W2,
                 e_b2, aW, ab, bW, bb, gW, gb, vW, vb):
    f32 = jnp.float32

    def wmap(i, bexp, bval):
        return (bexp[i], 0, 0)

    def xmap(i, bexp, bval):
        return (i, 0)

    o = pl.pallas_call(
        _mlp1_body,
        grid_spec=pltpu.PrefetchScalarGridSpec(
            num_scalar_prefetch=2,
            grid=(NB,),
            in_specs=[
                pl.BlockSpec((TB, D), xmap),
                pl.BlockSpec((TB, D), xmap),
                pl.BlockSpec((1, 2 * D, H), wmap),
                pl.BlockSpec((1, 1, H), wmap),
                pl.BlockSpec((1, 1, H), wmap),
                pl.BlockSpec((1, 1, H), wmap),
                pl.BlockSpec((1, H, 4 * D), wmap),
                pl.BlockSpec((1, 1, 4 * D), wmap),
            ],
            out_specs=pl.BlockSpec((TB, 4 * D), xmap),
        ),
        out_shape=jax.ShapeDtypeStruct((CAP, 4 * D), jnp.bfloat16),
    )(bexp, bval, xgh, xgx, e_W1, e_b1, e_ln_g, e_ln_b, e_W2, e_b2)

    y = pl.pallas_call(
        _mlp2_body,
        grid_spec=pltpu.PrefetchScalarGridSpec(
            num_scalar_prefetch=2,
            grid=(NB,),
            in_specs=[
                pl.BlockSpec((TB, 4 * D), xmap),
                pl.BlockSpec((1, D, D), wmap),
                pl.BlockSpec((1, 1, D), wmap),
                pl.BlockSpec((1, D, D), wmap),
                pl.BlockSpec((1, 1, D), wmap),
                pl.BlockSpec((1, D, D), wmap),
                pl.BlockSpec((1, 1, D), wmap),
                pl.BlockSpec((1, D, D), wmap),
                pl.BlockSpec((1, 1, D), wmap),
            ],
            out_specs=pl.BlockSpec((TB, 4 * D), xmap),
        ),
        out_shape=jax.ShapeDtypeStruct((CAP, 4 * D), f32),
    )(bexp, bval, o, aW, ab, bW, bb, gW, gb, vW, vb)
    return y


def _sc_combine_body(y_hbm, slot0_hbm, slot1_hbm, w0_hbm, w1_hbm,
                     oa_hbm, ob_hbm, og_hbm, ov_hbm,
                     idx0_v, idx1_v, w0_v, w1_v,
                     g0a, g0b, g1a, g1b, q0, q1, q2, q3, sem0, sem1):
    info = plsc.get_sparse_core_info()
    nc = info.num_cores
    wid = lax.axis_index("s") * nc + lax.axis_index("c")
    base = wid * TPW
    nch = TPW // _GCH
    pltpu.sync_copy(slot0_hbm.at[wid], idx0_v)
    pltpu.sync_copy(slot1_hbm.at[wid], idx1_v)
    pltpu.sync_copy(w0_hbm.at[pl.ds(base, TPW)], w0_v)
    pltpu.sync_copy(w1_hbm.at[pl.ds(base, TPW)], w1_v)

    buf0 = (g0a, g0b)
    buf1 = (g1a, g1b)
    quarters = (q0, q1, q2, q3)
    outs = (oa_hbm, ob_hbm, og_hbm, ov_hbm)

    def issue(c, par):
        i0 = idx0_v.at[c // 2, pl.ds((c % 2) * _GCH, _GCH)]
        i1 = idx1_v.at[c // 2, pl.ds((c % 2) * _GCH, _GCH)]
        return (pltpu.async_copy(y_hbm.at[i0], buf0[par], sem0),
                pltpu.async_copy(y_hbm.at[i1], buf1[par], sem1))

    pend = [None, None]
    pend[0] = issue(0, 0)
    for c in range(nch):
        if c + 1 < nch:
            pend[(c + 1) % 2] = issue(c + 1, (c + 1) % 2)
        for hnd in pend[c % 2]:
            hnd.wait()
        b0 = buf0[c % 2]
        b1 = buf1[c % 2]
        for t in range(_GCH):
            w0s = w0_v[c * _GCH + t]
            w1s = w1_v[c * _GCH + t]
            for q in range(4):
                qref = quarters[q]

                def body(j, carry, b0=b0, b1=b1, qref=qref, t=t, q=q,
                         w0s=w0s, w1s=w1s):
                    off = q * D + j * 16
                    r0 = b0[t, pl.ds(off, 16)]
                    r1 = b1[t, pl.ds(off, 16)]
                    qref[t, pl.ds(j * 16, 16)] = r0 * w0s + r1 * w1s
                    return carry

                lax.fori_loop(0, D // 16, body, 0)
        t0 = base + c * _GCH
        for q in range(4):
            pltpu.sync_copy(quarters[q], outs[q].at[pl.ds(t0, _GCH)])


def _sc_combine(y, slot0, slot1, w0, w1):
    f32 = jnp.float32
    mesh = plsc.VectorSubcoreMesh(core_axis_name="c", subcore_axis_name="s")
    f = functools.partial(
        pl.kernel,
        mesh=mesh,
        out_type=(
            jax.ShapeDtypeStruct((N, D), f32),
            jax.ShapeDtypeStruct((N, D), f32),
            jax.ShapeDtypeStruct((N, D), f32),
            jax.ShapeDtypeStruct((N, D), f32),
        ),
        scratch_types=[
            pltpu.VMEM((TPW // 8, 8), jnp.int32),
            pltpu.VMEM((TPW // 8, 8), jnp.int32),
            pltpu.VMEM((TPW, 16), f32),
            pltpu.VMEM((TPW, 16), f32),
            pltpu.VMEM((_GCH, 4 * D), f32),
            pltpu.VMEM((_GCH, 4 * D), f32),
            pltpu.VMEM((_GCH, 4 * D), f32),
            pltpu.VMEM((_GCH, 4 * D), f32),
            pltpu.VMEM((_GCH, D), f32),
            pltpu.VMEM((_GCH, D), f32),
            pltpu.VMEM((_GCH, D), f32),
            pltpu.VMEM((_GCH, D), f32),
            pltpu.SemaphoreType.DMA,
            pltpu.SemaphoreType.DMA,
        ],
    )(_sc_combine_body)
    return f(y, slot0, slot1, w0, w1)


def kernel(h, x, layer_idx, layer_emb_table, phase_emb_table, router_W1,
           router_b1, router_ln_g, router_ln_b, router_W2, router_b2,
           e_W1, e_b1, e_ln_g, e_ln_b, e_W2, e_b2,
           aW, ab, bW, bb, gW, gb, vW, vb):
    le = lax.dynamic_slice_in_dim(layer_emb_table, layer_idx, 1, 0)
    pe = phase_emb_table[0:1]
    a = router_W1[:D]
    b = router_W1[D:2 * D]
    rbias = (le @ router_W1[2 * D:2 * D + 32]
             + pe @ router_W1[2 * D + 32:]
             + router_b1.reshape(1, RD))

    oh1, oh2, w0, w1 = _router(
        h, x, a, b, rbias, router_ln_g.reshape(1, RD),
        router_ln_b.reshape(1, RD), router_W2, router_b2.reshape(1, E))

    slot0, slot1, bexp, bval = _metadata(oh1, oh2)
    slot0 = slot0.reshape(N)
    slot1 = slot1.reshape(N)
    bexp = bexp.reshape(NB)
    bval = bval.reshape(NB)

    xgh, xgx = _sc_scatter(h, x, slot0.reshape(NW, TPW // _SCH, _SCH),
                           slot1.reshape(NW, TPW // _SCH, _SCH))

    y = _grouped_mlp(xgh, xgx, bexp, bval, e_W1, e_b1.reshape(E, 1, H),
                     e_ln_g.reshape(E, 1, H), e_ln_b.reshape(E, 1, H), e_W2,
                     e_b2.reshape(E, 1, 4 * D), aW, ab.reshape(E, 1, D),
                     bW, bb.reshape(E, 1, D), gW, gb.reshape(E, 1, D),
                     vW, vb.reshape(E, 1, D))

    alpha, beta, gate, v_cand = _sc_combine(
        y, slot0.reshape(NW, TPW // 8, 8), slot1.reshape(NW, TPW // 8, 8),
        w0, w1)
    return (alpha, beta, gate, v_cand)
